# Initial kernel scaffold; baseline (speedup 1.0000x reference)
import jax
import jax.numpy as jnp
from jax.experimental import pallas as pl


def kernel(pos, flat_netpin, netpin_start, net_weights, net_mask):
    raise NotImplementedError("write your pallas kernel here")


# speedup vs baseline: 221.7230x; 221.7230x over previous
import functools

import jax
import jax.numpy as jnp
from jax import lax
from jax.experimental import pallas as pl
from jax.experimental.pallas import tpu as pltpu
from jax.experimental.pallas import tpu_sc as plsc

NW = 32
LANES = 16
WIN = 16384


def _body(npw, win, posx_hbm, posy_hbm, fnp_hbm, starts_hbm, w_hbm, out_hbm,
          starts_v, w_v, idx_v, xwin, ywin, accv, q0_ref, sem):
    groups = npw // LANES
    wid = lax.axis_index("c") * 16 + lax.axis_index("s")
    n0 = pl.multiple_of(wid * npw, npw)
    pltpu.sync_copy(starts_hbm.at[pl.ds(n0, npw + LANES)], starts_v)
    pltpu.sync_copy(w_hbm.at[pl.ds(n0, npw)], w_v)
    q0_ref[0] = jnp.int32(-win)

    iota = lax.iota(jnp.int32, LANES)
    big = jnp.int32(0x7FFFFFF0)

    def group(g, acc):
        base = g * LANES
        sv = plsc.load_gather(starts_v, [base + iota])
        ev = plsc.load_gather(starts_v, [base + 1 + iota])
        wv = plsc.load_gather(w_v, [base + iota])
        ln = ev - sv

        def wcond(st):
            t = st[0]
            return jnp.any(t < ln)

        def wbody(st):
            t, mnx, mxx, mny, mxy = st
            act = t < ln
            pin = sv + t
            q0 = q0_ref[0]
            m = act & (pin < q0 + win)

            def do_reload():
                nm = jnp.min(jnp.where(act, pin, big))
                q0n = pl.multiple_of(nm & jnp.int32(~127), 128)
                pltpu.sync_copy(fnp_hbm.at[pl.ds(q0n, win)], idx_v)
                cx = pltpu.async_copy(posx_hbm.at[idx_v], xwin, sem)
                cy = pltpu.async_copy(posy_hbm.at[idx_v], ywin, sem)
                cx.wait()
                cy.wait()
                q0_ref[0] = q0n

            pl.when(jnp.logical_not(jnp.any(m)))(do_reload)

            q0b = q0_ref[0]
            off = pin - q0b
            m2 = act & (off < win)
            vx = plsc.load_gather(xwin, [off], mask=m2)
            vy = plsc.load_gather(ywin, [off], mask=m2)
            mnx = jnp.where(m2, jnp.minimum(mnx, vx), mnx)
            mxx = jnp.where(m2, jnp.maximum(mxx, vx), mxx)
            mny = jnp.where(m2, jnp.minimum(mny, vy), mny)
            mxy = jnp.where(m2, jnp.maximum(mxy, vy), mxy)
            t = t + m2.astype(jnp.int32)
            return (t, mnx, mxx, mny, mxy)

        inf = jnp.full((LANES,), jnp.inf, jnp.float32)
        init = (jnp.zeros((LANES,), jnp.int32), inf, -inf, inf, -inf)
        _, mnx, mxx, mny, mxy = lax.while_loop(wcond, wbody, init)
        span = (mxx - mnx) + (mxy - mny)
        contrib = jnp.where(ln > 0, wv * span, jnp.zeros((LANES,), jnp.float32))
        return acc + contrib

    acc = lax.fori_loop(0, groups, group, jnp.zeros((LANES,), jnp.float32))
    accv[...] = acc
    pltpu.sync_copy(accv, out_hbm.at[wid])


def kernel(pos, flat_netpin, netpin_start, net_weights, net_mask):
    num_pins = flat_netpin.shape[0]
    num_nets = net_weights.shape[0]
    npw = -(-(-(-num_nets // NW)) // LANES) * LANES
    tot = NW * npw

    posx = pos[:num_pins]
    posy = pos[num_pins:]
    weff = jnp.where(net_mask, net_weights, jnp.float32(0.0))
    weff_pad = jnp.concatenate(
        [weff, jnp.zeros((tot - num_nets,), jnp.float32)])
    starts_pad = jnp.concatenate([
        netpin_start.astype(jnp.int32),
        jnp.full((tot + LANES - num_nets - 1,), num_pins, jnp.int32),
    ])
    fp = -((num_pins + WIN) // -128) * 128
    fnp_pad = jnp.concatenate(
        [flat_netpin, jnp.zeros((fp - num_pins,), jnp.int32)])

    mesh = plsc.VectorSubcoreMesh(
        core_axis_name="c", subcore_axis_name="s", num_cores=2,
        num_subcores=16)
    grid_kernel = pl.kernel(
        functools.partial(_body, npw, WIN),
        out_type=jax.ShapeDtypeStruct((NW, LANES), jnp.float32),
        mesh=mesh,
        compiler_params=pltpu.CompilerParams(needs_layout_passes=False),
        scratch_types=[
            pltpu.VMEM((npw + LANES,), jnp.int32),
            pltpu.VMEM((npw,), jnp.float32),
            pltpu.VMEM((WIN,), jnp.int32),
            pltpu.VMEM((WIN,), jnp.float32),
            pltpu.VMEM((WIN,), jnp.float32),
            pltpu.VMEM((LANES,), jnp.float32),
            pltpu.SMEM((1,), jnp.int32),
            pltpu.SemaphoreType.DMA,
        ],
    )
    partials = grid_kernel(posx, posy, fnp_pad, starts_pad, weff_pad)
    return jnp.sum(partials)
